# baseline (speedup 1.0000x reference)
import functools

import jax
import jax.numpy as jnp
from jax import lax
from jax.experimental import pallas as pl
from jax.experimental.pallas import tpu as pltpu
from jax.experimental.pallas import tpu_sc as plsc

B = 16
H_NUM = 3
W_NUM = 9
GRID = 20
DIM = 1024
L = 64
T = H_NUM * W_NUM + 1

NC = 2
NS = 16


def _sc_gather3(row_embed, col_embed, type_embed):
    mesh = plsc.ScalarSubcoreMesh(axis_name="c", num_cores=1)

    @functools.partial(
        pl.kernel,
        mesh=mesh,
        out_type=jax.ShapeDtypeStruct((3 * T, DIM), jnp.float32),
        scratch_types=[pltpu.SemaphoreType.DMA],
    )
    def sc_fn(row_hbm, col_hbm, typ_hbm, c3_hbm, sem):
        copies = []
        for t in range(T):
            y = GRID - 1 if t == T - 1 else t // W_NUM
            x = GRID - 1 if t == T - 1 else t % W_NUM
            m = 1 if t == T - 1 else 0
            copies.append(pltpu.async_copy(row_hbm.at[y], c3_hbm.at[3 * t], sem))
            copies.append(pltpu.async_copy(col_hbm.at[x], c3_hbm.at[3 * t + 1], sem))
            copies.append(pltpu.async_copy(typ_hbm.at[m], c3_hbm.at[3 * t + 2], sem))
        for c in copies:
            c.wait()

    return sc_fn(row_embed, col_embed, type_embed)


def _tc_body(it_ref, local_ref, c3_ref, img_ref, out_ref):
    b = pl.program_id(0)
    it = it_ref[b]
    img = jnp.where(it == 0, img_ref[0, :], img_ref[1, :])
    tmp = local_ref[0] + img[None, :]
    for t in range(T):
        comb = c3_ref[3 * t, :] + c3_ref[3 * t + 1, :] + c3_ref[3 * t + 2, :]
        out_ref[0, t * L:(t + 1) * L, :] = tmp + comb[None, :]


def kernel(image_time, local_pos, row_embed, col_embed, image_embed, type_embed):
    it32 = image_time.astype(jnp.int32)
    c3 = _sc_gather3(row_embed, col_embed, type_embed)

    out = pl.pallas_call(
        _tc_body,
        grid=(B,),
        in_specs=[
            pl.BlockSpec(memory_space=pltpu.SMEM),
            pl.BlockSpec((1, L, DIM), lambda b: (0, 0, 0)),
            pl.BlockSpec((3 * T, DIM), lambda b: (0, 0)),
            pl.BlockSpec((2, DIM), lambda b: (0, 0)),
        ],
        out_specs=pl.BlockSpec((1, T * L, DIM), lambda b: (b, 0, 0)),
        out_shape=jax.ShapeDtypeStruct((B, T * L, DIM), jnp.float32),
    )(it32, local_pos, c3, image_embed)
    return out


# speedup vs baseline: 1.1162x; 1.1162x over previous
import functools

import jax
import jax.numpy as jnp
from jax import lax
from jax.experimental import pallas as pl
from jax.experimental.pallas import tpu as pltpu
from jax.experimental.pallas import tpu_sc as plsc

B = 16
H_NUM = 3
W_NUM = 9
GRID = 20
DIM = 1024
L = 64
T = H_NUM * W_NUM + 1

NC = 2
NS = 16


def _sc_comb(row_embed, col_embed, type_embed):
    mesh = plsc.VectorSubcoreMesh(core_axis_name="c", subcore_axis_name="s",
                                  num_cores=1)

    @functools.partial(
        pl.kernel,
        mesh=mesh,
        out_type=jax.ShapeDtypeStruct((T, DIM), jnp.float32),
        scratch_types=[
            pltpu.VMEM((2, DIM), jnp.float32),
            pltpu.VMEM((2, DIM), jnp.float32),
            pltpu.VMEM((2, DIM), jnp.float32),
            pltpu.VMEM((DIM,), jnp.float32),
            pltpu.SemaphoreType.DMA,
            pltpu.SemaphoreType.DMA,
            pltpu.SemaphoreType.DMA,
            pltpu.SemaphoreType.DMA,
            pltpu.SemaphoreType.DMA,
            pltpu.SemaphoreType.DMA,
        ],
    )
    def sc_fn(row_hbm, col_hbm, typ_hbm, comb_hbm, ry_v, rx_v, rm_v, acc_v,
              s0a, s1a, s2a, s0b, s1b, s2b):
        wid = lax.axis_index("s")

        def idxs(r):
            is_thumb = r == T - 1
            y = jnp.where(is_thumb, GRID - 1, r // W_NUM)
            x = jnp.where(is_thumb, GRID - 1, r % W_NUM)
            m = jnp.where(is_thumb, 1, 0)
            return y, x, m

        r0 = wid
        r1 = jnp.minimum(wid + NS, T - 1)
        y0, x0, m0 = idxs(r0)
        y1, x1, m1 = idxs(r1)
        ga = (pltpu.async_copy(row_hbm.at[y0], ry_v.at[0], s0a),
              pltpu.async_copy(col_hbm.at[x0], rx_v.at[0], s1a),
              pltpu.async_copy(typ_hbm.at[m0], rm_v.at[0], s2a))
        gb = (pltpu.async_copy(row_hbm.at[y1], ry_v.at[1], s0b),
              pltpu.async_copy(col_hbm.at[x1], rx_v.at[1], s1b),
              pltpu.async_copy(typ_hbm.at[m1], rm_v.at[1], s2b))

        UNROLL = 4

        def reduce_to(k, dst_r):
            def chunk(j, _):
                for u in range(UNROLL):
                    d = pl.ds((j * UNROLL + u) * 16, 16)
                    acc_v[d] = ry_v[k, d] + rx_v[k, d] + rm_v[k, d]
                return 0

            lax.fori_loop(0, DIM // 16 // UNROLL, chunk, 0)
            pltpu.sync_copy(acc_v, comb_hbm.at[dst_r])

        for g in ga:
            g.wait()
        reduce_to(0, r0)
        for g in gb:
            g.wait()

        @pl.when(wid < T - NS)
        def _():
            reduce_to(1, wid + NS)

    return sc_fn(row_embed, col_embed, type_embed)


def _tc_body(it_ref, local_ref, comb_ref, img_ref, out_ref):
    b = pl.program_id(0)
    it = it_ref[b]
    img = jnp.where(it == 0, img_ref[0, :], img_ref[1, :])
    tmp = local_ref[0] + img[None, :]
    for t in range(T):
        out_ref[0, t * L:(t + 1) * L, :] = tmp + comb_ref[t, :][None, :]


def kernel(image_time, local_pos, row_embed, col_embed, image_embed, type_embed):
    it32 = image_time.astype(jnp.int32)
    comb = _sc_comb(row_embed, col_embed, type_embed)

    out = pl.pallas_call(
        _tc_body,
        grid=(B,),
        in_specs=[
            pl.BlockSpec(memory_space=pltpu.SMEM),
            pl.BlockSpec((1, L, DIM), lambda b: (0, 0, 0)),
            pl.BlockSpec((T, DIM), lambda b: (0, 0)),
            pl.BlockSpec((2, DIM), lambda b: (0, 0)),
        ],
        out_specs=pl.BlockSpec((1, T * L, DIM), lambda b: (b, 0, 0)),
        out_shape=jax.ShapeDtypeStruct((B, T * L, DIM), jnp.float32),
    )(it32, local_pos, comb, image_embed)
    return out
